# baseline (device time: 82055 ns/iter reference)
import jax
import jax.numpy as jnp
from jax import lax
from jax.experimental import pallas as pl
from jax.experimental.pallas import tpu as pltpu

N_DEV = 16
B, SQ, SKV, HQ, DH = 2, 128, 128, 64, 64
H_LOC = HQ // N_DEV
D_MODEL = 512
ROWS = B * SQ


def kernel(x, Wq, K_ext, V_ext, Wo):
    my = lax.axis_index("i")
    K_loc = jnp.transpose(
        lax.dynamic_slice_in_dim(K_ext, my * H_LOC, H_LOC, axis=2), (0, 2, 1, 3)
    )
    V_loc = jnp.transpose(
        lax.dynamic_slice_in_dim(V_ext, my * H_LOC, H_LOC, axis=2), (0, 2, 1, 3)
    )

    def body(x_ref, wq_ref, k_ref, v_ref, wo_ref, out_ref,
             gather_buf, send_sems, recv_sems):
        my_pos = lax.axis_index("i")
        left = jnp.remainder(my_pos - 1, N_DEV)
        right = jnp.remainder(my_pos + 1, N_DEV)

        barrier_sem = pltpu.get_barrier_semaphore()
        for nbr in (left, right):
            pl.semaphore_signal(
                barrier_sem, inc=1,
                device_id=(nbr,), device_id_type=pl.DeviceIdType.MESH,
            )
        pl.semaphore_wait(barrier_sem, 2)

        wq = wq_ref[:, :].astype(jnp.bfloat16)
        wo = wo_ref[:, :].astype(jnp.bfloat16)
        parts = []
        for b in range(B):
            xb = x_ref[b, :, :].astype(jnp.bfloat16)
            q_b = lax.dot_general(
                xb, wq, (((1,), (0,)), ((), ())),
                preferred_element_type=jnp.float32,
            )
            ctx_h = []
            for h in range(H_LOC):
                q_bh = q_b[:, h * DH:(h + 1) * DH].astype(jnp.bfloat16)
                k_bh = k_ref[b, h, :, :].astype(jnp.bfloat16)
                s = lax.dot_general(
                    q_bh, k_bh, (((1,), (1,)), ((), ())),
                    preferred_element_type=jnp.float32,
                ) * 0.125
                s = s - jnp.max(s, axis=1, keepdims=True)
                w = jnp.exp(s)
                w = (w / jnp.sum(w, axis=1, keepdims=True)).astype(jnp.bfloat16)
                v_bh = v_ref[b, h, :, :].astype(jnp.bfloat16)
                ctx_h.append(lax.dot_general(
                    w, v_bh, (((1,), (0,)), ((), ())),
                    preferred_element_type=jnp.float32,
                ))
            ctx_b = jnp.concatenate(ctx_h, axis=1).astype(jnp.bfloat16)
            parts.append(lax.dot_general(
                ctx_b, wo, (((1,), (0,)), ((), ())),
                preferred_element_type=jnp.float32,
            ).astype(jnp.bfloat16))
        partial = jnp.concatenate(parts, axis=0)
        gather_buf[pl.ds(my_pos * ROWS, ROWS), :] = partial

        for hop in range(N_DEV - 1):
            src = jnp.remainder(my_pos - hop, N_DEV)
            rdma = pltpu.make_async_remote_copy(
                src_ref=gather_buf.at[pl.ds(src * ROWS, ROWS), :],
                dst_ref=gather_buf.at[pl.ds(src * ROWS, ROWS), :],
                send_sem=send_sems.at[hop],
                recv_sem=recv_sems.at[hop],
                device_id=(right,),
                device_id_type=pl.DeviceIdType.MESH,
            )
            rdma.start()
            rdma.wait()

        acc = gather_buf[0:ROWS, :].astype(jnp.float32)
        for s in range(1, N_DEV):
            acc = acc + gather_buf[s * ROWS:(s + 1) * ROWS, :].astype(jnp.float32)
        out_ref[0, :, :] = acc[:SQ, :]
        out_ref[1, :, :] = acc[SQ:, :]

    return pl.pallas_call(
        body,
        out_shape=jax.ShapeDtypeStruct((B, SQ, D_MODEL), jnp.float32),
        in_specs=[pl.BlockSpec(memory_space=pltpu.VMEM)] * 5,
        out_specs=pl.BlockSpec(memory_space=pltpu.VMEM),
        scratch_shapes=[
            pltpu.VMEM((N_DEV * ROWS, D_MODEL), jnp.bfloat16),
            pltpu.SemaphoreType.DMA((N_DEV - 1,)),
            pltpu.SemaphoreType.DMA((N_DEV - 1,)),
        ],
        compiler_params=pltpu.CompilerParams(collective_id=0),
    )(x, Wq, K_loc, V_loc, Wo)


# device time: 24839 ns/iter; 3.3035x vs baseline; 3.3035x over previous
import jax
import jax.numpy as jnp
from jax import lax
from jax.experimental import pallas as pl
from jax.experimental.pallas import tpu as pltpu

N_DEV = 16
B, SQ, SKV, HQ, DH = 2, 128, 128, 64, 64
H_LOC = HQ // N_DEV
D_MODEL = 512
ROWS = B * SQ
CHUNK = ROWS // N_DEV


def kernel(x, Wq, K_ext, V_ext, Wo):
    my = lax.axis_index("i")
    K_loc = jnp.transpose(
        lax.dynamic_slice_in_dim(K_ext, my * H_LOC, H_LOC, axis=2), (0, 2, 1, 3)
    )
    V_loc = jnp.transpose(
        lax.dynamic_slice_in_dim(V_ext, my * H_LOC, H_LOC, axis=2), (0, 2, 1, 3)
    )

    def body(x_ref, wq_ref, k_ref, v_ref, wo_ref, out_ref,
             part_buf, rs_buf, gather_buf,
             send1, recv1, send2, recv2):
        my_pos = lax.axis_index("i")

        barrier_sem = pltpu.get_barrier_semaphore()
        for k in range(1, N_DEV):
            pl.semaphore_signal(
                barrier_sem, inc=1,
                device_id=(jnp.remainder(my_pos + k, N_DEV),),
                device_id_type=pl.DeviceIdType.MESH,
            )
        pl.semaphore_wait(barrier_sem, N_DEV - 1)

        wq = wq_ref[:, :].astype(jnp.bfloat16)
        wo = wo_ref[:, :].astype(jnp.bfloat16)
        parts = []
        for b in range(B):
            xb = x_ref[b, :, :].astype(jnp.bfloat16)
            q_b = lax.dot_general(
                xb, wq, (((1,), (0,)), ((), ())),
                preferred_element_type=jnp.float32,
            )
            ctx_h = []
            for h in range(H_LOC):
                q_bh = q_b[:, h * DH:(h + 1) * DH].astype(jnp.bfloat16)
                k_bh = k_ref[b, h, :, :].astype(jnp.bfloat16)
                s = lax.dot_general(
                    q_bh, k_bh, (((1,), (1,)), ((), ())),
                    preferred_element_type=jnp.float32,
                ) * 0.125
                s = s - jnp.max(s, axis=1, keepdims=True)
                w = jnp.exp(s)
                w = (w / jnp.sum(w, axis=1, keepdims=True)).astype(jnp.bfloat16)
                v_bh = v_ref[b, h, :, :].astype(jnp.bfloat16)
                ctx_h.append(lax.dot_general(
                    w, v_bh, (((1,), (0,)), ((), ())),
                    preferred_element_type=jnp.float32,
                ))
            ctx_b = jnp.concatenate(ctx_h, axis=1).astype(jnp.bfloat16)
            parts.append(lax.dot_general(
                ctx_b, wo, (((1,), (0,)), ((), ())),
                preferred_element_type=jnp.float32,
            ).astype(jnp.bfloat16))
        part_buf[:, :] = jnp.concatenate(parts, axis=0)

        sends = []
        for k in range(1, N_DEV):
            tgt = jnp.remainder(my_pos + k, N_DEV)
            rdma = pltpu.make_async_remote_copy(
                src_ref=part_buf.at[pl.ds(tgt * CHUNK, CHUNK), :],
                dst_ref=rs_buf.at[pl.ds(my_pos * CHUNK, CHUNK), :],
                send_sem=send1.at[k - 1],
                recv_sem=recv1.at[k - 1],
                device_id=(tgt,),
                device_id_type=pl.DeviceIdType.MESH,
            )
            rdma.start()
            sends.append(rdma)
        rs_buf[pl.ds(my_pos * CHUNK, CHUNK), :] = (
            part_buf[pl.ds(my_pos * CHUNK, CHUNK), :]
        )
        for k in range(1, N_DEV):
            src_dev = jnp.remainder(my_pos - k, N_DEV)
            pltpu.make_async_remote_copy(
                src_ref=part_buf.at[pl.ds(0, CHUNK), :],
                dst_ref=rs_buf.at[pl.ds(src_dev * CHUNK, CHUNK), :],
                send_sem=send1.at[k - 1],
                recv_sem=recv1.at[k - 1],
                device_id=(src_dev,),
                device_id_type=pl.DeviceIdType.MESH,
            ).wait_recv()

        acc = rs_buf[0:CHUNK, :].astype(jnp.float32)
        for s in range(1, N_DEV):
            acc = acc + rs_buf[s * CHUNK:(s + 1) * CHUNK, :].astype(jnp.float32)
        gather_buf[pl.ds(my_pos * CHUNK, CHUNK), :] = acc.astype(jnp.bfloat16)

        for k in range(1, N_DEV):
            tgt = jnp.remainder(my_pos + k, N_DEV)
            rdma = pltpu.make_async_remote_copy(
                src_ref=gather_buf.at[pl.ds(my_pos * CHUNK, CHUNK), :],
                dst_ref=gather_buf.at[pl.ds(my_pos * CHUNK, CHUNK), :],
                send_sem=send2.at[k - 1],
                recv_sem=recv2.at[k - 1],
                device_id=(tgt,),
                device_id_type=pl.DeviceIdType.MESH,
            )
            rdma.start()
            sends.append(rdma)
        for k in range(1, N_DEV):
            src_dev = jnp.remainder(my_pos - k, N_DEV)
            pltpu.make_async_remote_copy(
                src_ref=gather_buf.at[pl.ds(0, CHUNK), :],
                dst_ref=gather_buf.at[pl.ds(src_dev * CHUNK, CHUNK), :],
                send_sem=send2.at[k - 1],
                recv_sem=recv2.at[k - 1],
                device_id=(src_dev,),
                device_id_type=pl.DeviceIdType.MESH,
            ).wait_recv()

        out_ref[0, :, :] = gather_buf[0:SQ, :].astype(jnp.float32)
        out_ref[1, :, :] = gather_buf[SQ:ROWS, :].astype(jnp.float32)

        for rdma in sends:
            rdma.wait_send()

    return pl.pallas_call(
        body,
        out_shape=jax.ShapeDtypeStruct((B, SQ, D_MODEL), jnp.float32),
        in_specs=[pl.BlockSpec(memory_space=pltpu.VMEM)] * 5,
        out_specs=pl.BlockSpec(memory_space=pltpu.VMEM),
        scratch_shapes=[
            pltpu.VMEM((ROWS, D_MODEL), jnp.bfloat16),
            pltpu.VMEM((ROWS, D_MODEL), jnp.bfloat16),
            pltpu.VMEM((ROWS, D_MODEL), jnp.bfloat16),
            pltpu.SemaphoreType.DMA((N_DEV - 1,)),
            pltpu.SemaphoreType.DMA((N_DEV - 1,)),
            pltpu.SemaphoreType.DMA((N_DEV - 1,)),
            pltpu.SemaphoreType.DMA((N_DEV - 1,)),
        ],
        compiler_params=pltpu.CompilerParams(collective_id=0),
    )(x, Wq, K_loc, V_loc, Wo)


# device time: 21834 ns/iter; 3.7581x vs baseline; 1.1376x over previous
import jax
import jax.numpy as jnp
from jax import lax
from jax.experimental import pallas as pl
from jax.experimental.pallas import tpu as pltpu

N_DEV = 16
B, SQ, SKV, HQ, DH = 2, 128, 128, 64, 64
H_LOC = HQ // N_DEV
D_MODEL = 512
ROWS = B * SQ
CHUNK = ROWS // N_DEV


def kernel(x, Wq, K_ext, V_ext, Wo):
    my = lax.axis_index("i")
    K_loc = jnp.transpose(
        lax.dynamic_slice_in_dim(K_ext, my * H_LOC, H_LOC, axis=2), (0, 2, 1, 3)
    )
    V_loc = jnp.transpose(
        lax.dynamic_slice_in_dim(V_ext, my * H_LOC, H_LOC, axis=2), (0, 2, 1, 3)
    )

    def body(x_ref, wq_ref, k_ref, v_ref, wo_ref, out_ref,
             part_buf, rs_buf, gather_buf,
             send1, recv1, send2, recv2):
        my_pos = lax.axis_index("i")

        barrier_sem = pltpu.get_barrier_semaphore()
        for k in range(1, N_DEV):
            pl.semaphore_signal(
                barrier_sem, inc=1,
                device_id=(jnp.remainder(my_pos + k, N_DEV),),
                device_id_type=pl.DeviceIdType.MESH,
            )

        wq = wq_ref[:, :].astype(jnp.bfloat16)
        wo = wo_ref[:, :].astype(jnp.bfloat16)
        parts = []
        for b in range(B):
            xb = x_ref[b, :, :].astype(jnp.bfloat16)
            q_b = lax.dot_general(
                xb, wq, (((1,), (0,)), ((), ())),
                preferred_element_type=jnp.float32,
            )
            ctx_h = []
            for h in range(H_LOC):
                q_bh = q_b[:, h * DH:(h + 1) * DH].astype(jnp.bfloat16)
                k_bh = k_ref[b, h, :, :].astype(jnp.bfloat16)
                s = lax.dot_general(
                    q_bh, k_bh, (((1,), (1,)), ((), ())),
                    preferred_element_type=jnp.float32,
                ) * 0.125
                s = s - jnp.max(s, axis=1, keepdims=True)
                w = jnp.exp(s)
                w = (w / jnp.sum(w, axis=1, keepdims=True)).astype(jnp.bfloat16)
                v_bh = v_ref[b, h, :, :].astype(jnp.bfloat16)
                ctx_h.append(lax.dot_general(
                    w, v_bh, (((1,), (0,)), ((), ())),
                    preferred_element_type=jnp.float32,
                ))
            ctx_b = jnp.concatenate(ctx_h, axis=1).astype(jnp.bfloat16)
            parts.append(lax.dot_general(
                ctx_b, wo, (((1,), (0,)), ((), ())),
                preferred_element_type=jnp.float32,
            ).astype(jnp.bfloat16))
        part_buf[:, :] = jnp.concatenate(parts, axis=0)

        pl.semaphore_wait(barrier_sem, N_DEV - 1)

        sends = []
        for k in range(1, N_DEV):
            tgt = jnp.remainder(my_pos + k, N_DEV)
            rdma = pltpu.make_async_remote_copy(
                src_ref=part_buf.at[pl.ds(tgt * CHUNK, CHUNK), :],
                dst_ref=rs_buf.at[pl.ds(my_pos * CHUNK, CHUNK), :],
                send_sem=send1.at[k - 1],
                recv_sem=recv1.at[k - 1],
                device_id=(tgt,),
                device_id_type=pl.DeviceIdType.MESH,
            )
            rdma.start()
            sends.append(rdma)
        rs_buf[pl.ds(my_pos * CHUNK, CHUNK), :] = (
            part_buf[pl.ds(my_pos * CHUNK, CHUNK), :]
        )
        for k in range(1, N_DEV):
            src_dev = jnp.remainder(my_pos - k, N_DEV)
            pltpu.make_async_remote_copy(
                src_ref=part_buf.at[pl.ds(0, CHUNK), :],
                dst_ref=rs_buf.at[pl.ds(src_dev * CHUNK, CHUNK), :],
                send_sem=send1.at[k - 1],
                recv_sem=recv1.at[k - 1],
                device_id=(src_dev,),
                device_id_type=pl.DeviceIdType.MESH,
            ).wait_recv()

        acc = rs_buf[0:CHUNK, :].astype(jnp.float32)
        for s in range(1, N_DEV):
            acc = acc + rs_buf[s * CHUNK:(s + 1) * CHUNK, :].astype(jnp.float32)
        gather_buf[pl.ds(my_pos * CHUNK, CHUNK), :] = acc.astype(jnp.bfloat16)

        for k in range(1, N_DEV):
            tgt = jnp.remainder(my_pos + k, N_DEV)
            rdma = pltpu.make_async_remote_copy(
                src_ref=gather_buf.at[pl.ds(my_pos * CHUNK, CHUNK), :],
                dst_ref=gather_buf.at[pl.ds(my_pos * CHUNK, CHUNK), :],
                send_sem=send2.at[k - 1],
                recv_sem=recv2.at[k - 1],
                device_id=(tgt,),
                device_id_type=pl.DeviceIdType.MESH,
            )
            rdma.start()
            sends.append(rdma)
        out_ref[pl.ds(my_pos * CHUNK, CHUNK), :] = acc
        for k in range(1, N_DEV):
            src_dev = jnp.remainder(my_pos - k, N_DEV)
            pltpu.make_async_remote_copy(
                src_ref=gather_buf.at[pl.ds(0, CHUNK), :],
                dst_ref=gather_buf.at[pl.ds(src_dev * CHUNK, CHUNK), :],
                send_sem=send2.at[k - 1],
                recv_sem=recv2.at[k - 1],
                device_id=(src_dev,),
                device_id_type=pl.DeviceIdType.MESH,
            ).wait_recv()
            out_ref[pl.ds(src_dev * CHUNK, CHUNK), :] = (
                gather_buf[pl.ds(src_dev * CHUNK, CHUNK), :].astype(jnp.float32)
            )

        for rdma in sends:
            rdma.wait_send()

    out = pl.pallas_call(
        body,
        out_shape=jax.ShapeDtypeStruct((ROWS, D_MODEL), jnp.float32),
        in_specs=[pl.BlockSpec(memory_space=pltpu.VMEM)] * 5,
        out_specs=pl.BlockSpec(memory_space=pltpu.VMEM),
        scratch_shapes=[
            pltpu.VMEM((ROWS, D_MODEL), jnp.bfloat16),
            pltpu.VMEM((ROWS, D_MODEL), jnp.bfloat16),
            pltpu.VMEM((ROWS, D_MODEL), jnp.bfloat16),
            pltpu.SemaphoreType.DMA((N_DEV - 1,)),
            pltpu.SemaphoreType.DMA((N_DEV - 1,)),
            pltpu.SemaphoreType.DMA((N_DEV - 1,)),
            pltpu.SemaphoreType.DMA((N_DEV - 1,)),
        ],
        compiler_params=pltpu.CompilerParams(collective_id=0),
    )(x, Wq, K_loc, V_loc, Wo)
    return out.reshape(B, SQ, D_MODEL)


# device time: 20899 ns/iter; 3.9263x vs baseline; 1.0447x over previous
import jax
import jax.numpy as jnp
from jax import lax
from jax.experimental import pallas as pl
from jax.experimental.pallas import tpu as pltpu

N_DEV = 16
B, SQ, SKV, HQ, DH = 2, 128, 128, 64, 64
H_LOC = HQ // N_DEV
D_MODEL = 512
ROWS = B * SQ
CHUNK = ROWS // N_DEV


def kernel(x, Wq, K_ext, V_ext, Wo):
    my = lax.axis_index("i")
    K_loc = jnp.transpose(
        lax.dynamic_slice_in_dim(K_ext, my * H_LOC, H_LOC, axis=2), (0, 2, 1, 3)
    )
    V_loc = jnp.transpose(
        lax.dynamic_slice_in_dim(V_ext, my * H_LOC, H_LOC, axis=2), (0, 2, 1, 3)
    )

    def body(x_ref, wq_ref, k_ref, v_ref, wo_ref, out_ref,
             part_buf, rs_buf, gather_buf,
             send1, recv1, send2, recv2):
        my_pos = lax.axis_index("i")

        barrier_sem = pltpu.get_barrier_semaphore()
        for k in range(1, N_DEV):
            pl.semaphore_signal(
                barrier_sem, inc=1,
                device_id=(jnp.remainder(my_pos + k, N_DEV),),
                device_id_type=pl.DeviceIdType.MESH,
            )

        wq = wq_ref[:, :].astype(jnp.bfloat16)
        wo = wo_ref[:, :].astype(jnp.bfloat16)
        for b in range(B):
            xb = x_ref[b, :, :].astype(jnp.bfloat16)
            q_b = lax.dot_general(
                xb, wq, (((1,), (0,)), ((), ())),
                preferred_element_type=jnp.float32,
            )
            ctx_h = []
            for h in range(H_LOC):
                q_bh = q_b[:, h * DH:(h + 1) * DH].astype(jnp.bfloat16)
                k_bh = k_ref[b, h, :, :].astype(jnp.bfloat16)
                s = lax.dot_general(
                    q_bh, k_bh, (((1,), (1,)), ((), ())),
                    preferred_element_type=jnp.float32,
                ) * 0.125
                s = s - jnp.max(s, axis=1, keepdims=True)
                w = jnp.exp(s)
                w = (w / jnp.sum(w, axis=1, keepdims=True)).astype(jnp.bfloat16)
                v_bh = v_ref[b, h, :, :].astype(jnp.bfloat16)
                ctx_h.append(lax.dot_general(
                    w, v_bh, (((1,), (0,)), ((), ())),
                    preferred_element_type=jnp.float32,
                ))
            ctx_b = jnp.concatenate(ctx_h, axis=1).astype(jnp.bfloat16)
            p_b = lax.dot_general(
                ctx_b, wo, (((1,), (0,)), ((), ())),
                preferred_element_type=jnp.float32,
            ).astype(jnp.bfloat16)
            part_buf[pl.ds(b * SQ, SQ), :] = p_b

            if b == 0:
                pl.semaphore_wait(barrier_sem, N_DEV - 1)
            lo, hi = b * (N_DEV // B), (b + 1) * (N_DEV // B)
            for k in range(1, N_DEV):
                tgt = jnp.remainder(my_pos + k, N_DEV)

                @pl.when(jnp.logical_and(tgt >= lo, tgt < hi))
                def _(k=k, tgt=tgt):
                    pltpu.make_async_remote_copy(
                        src_ref=part_buf.at[pl.ds(tgt * CHUNK, CHUNK), :],
                        dst_ref=rs_buf.at[pl.ds(my_pos * CHUNK, CHUNK), :],
                        send_sem=send1.at[k - 1],
                        recv_sem=recv1.at[k - 1],
                        device_id=(tgt,),
                        device_id_type=pl.DeviceIdType.MESH,
                    ).start()

        sends = []
        rs_buf[pl.ds(my_pos * CHUNK, CHUNK), :] = (
            part_buf[pl.ds(my_pos * CHUNK, CHUNK), :]
        )
        for k in range(1, N_DEV):
            src_dev = jnp.remainder(my_pos - k, N_DEV)
            pltpu.make_async_remote_copy(
                src_ref=part_buf.at[pl.ds(0, CHUNK), :],
                dst_ref=rs_buf.at[pl.ds(src_dev * CHUNK, CHUNK), :],
                send_sem=send1.at[k - 1],
                recv_sem=recv1.at[k - 1],
                device_id=(src_dev,),
                device_id_type=pl.DeviceIdType.MESH,
            ).wait_recv()

        acc = rs_buf[0:CHUNK, :].astype(jnp.float32)
        for s in range(1, N_DEV):
            acc = acc + rs_buf[s * CHUNK:(s + 1) * CHUNK, :].astype(jnp.float32)
        gather_buf[pl.ds(my_pos * CHUNK, CHUNK), :] = acc.astype(jnp.bfloat16)

        for k in range(1, N_DEV):
            tgt = jnp.remainder(my_pos + k, N_DEV)
            rdma = pltpu.make_async_remote_copy(
                src_ref=gather_buf.at[pl.ds(my_pos * CHUNK, CHUNK), :],
                dst_ref=gather_buf.at[pl.ds(my_pos * CHUNK, CHUNK), :],
                send_sem=send2.at[k - 1],
                recv_sem=recv2.at[k - 1],
                device_id=(tgt,),
                device_id_type=pl.DeviceIdType.MESH,
            )
            rdma.start()
            sends.append(rdma)
        out_ref[pl.ds(my_pos * CHUNK, CHUNK), :] = acc
        for k in range(1, N_DEV):
            src_dev = jnp.remainder(my_pos - k, N_DEV)
            pltpu.make_async_remote_copy(
                src_ref=gather_buf.at[pl.ds(0, CHUNK), :],
                dst_ref=gather_buf.at[pl.ds(src_dev * CHUNK, CHUNK), :],
                send_sem=send2.at[k - 1],
                recv_sem=recv2.at[k - 1],
                device_id=(src_dev,),
                device_id_type=pl.DeviceIdType.MESH,
            ).wait_recv()
            out_ref[pl.ds(src_dev * CHUNK, CHUNK), :] = (
                gather_buf[pl.ds(src_dev * CHUNK, CHUNK), :].astype(jnp.float32)
            )

        for k in range(1, N_DEV):
            pltpu.make_async_remote_copy(
                src_ref=part_buf.at[pl.ds(0, CHUNK), :],
                dst_ref=rs_buf.at[pl.ds(0, CHUNK), :],
                send_sem=send1.at[k - 1],
                recv_sem=recv1.at[k - 1],
                device_id=(my_pos,),
                device_id_type=pl.DeviceIdType.MESH,
            ).wait_send()
        for rdma in sends:
            rdma.wait_send()

    out = pl.pallas_call(
        body,
        out_shape=jax.ShapeDtypeStruct((ROWS, D_MODEL), jnp.float32),
        in_specs=[pl.BlockSpec(memory_space=pltpu.VMEM)] * 5,
        out_specs=pl.BlockSpec(memory_space=pltpu.VMEM),
        scratch_shapes=[
            pltpu.VMEM((ROWS, D_MODEL), jnp.bfloat16),
            pltpu.VMEM((ROWS, D_MODEL), jnp.bfloat16),
            pltpu.VMEM((ROWS, D_MODEL), jnp.bfloat16),
            pltpu.SemaphoreType.DMA((N_DEV - 1,)),
            pltpu.SemaphoreType.DMA((N_DEV - 1,)),
            pltpu.SemaphoreType.DMA((N_DEV - 1,)),
            pltpu.SemaphoreType.DMA((N_DEV - 1,)),
        ],
        compiler_params=pltpu.CompilerParams(collective_id=0),
    )(x, Wq, K_loc, V_loc, Wo)
    return out.reshape(B, SQ, D_MODEL)
